# Initial kernel scaffold; baseline (speedup 1.0000x reference)
import jax
import jax.numpy as jnp
from jax.experimental import pallas as pl


def kernel(boxes, scores, labels, hidden_states):
    raise NotImplementedError("write your pallas kernel here")


# speedup vs baseline: 273.3180x; 273.3180x over previous
import jax
import jax.numpy as jnp
from jax import lax
from jax.experimental import pallas as pl

_N = 5000
_B = 512
_P = 5120
_T = _P // _B
_IOU_THRESH = 0.5
_SCORE_THRESH = 0.2
_NEG = -1e30
_MAXI = 15
_HUMAN = 0


def _nms_topk_kernel(x1r, y1r, x2r, y2r, x1c, y1c, x2c, y2c, sc, lb, out_ref):
    X1r, Y1r, X2r, Y2r = x1r[...], y1r[...], x2r[...], y2r[...]
    X1c, Y1c, X2c, Y2c = x1c[...], y1c[...], x2c[...], y2c[...]
    S = sc[...]
    L = lb[...]

    Ar = jnp.maximum(X2r - X1r, 0.0) * jnp.maximum(Y2r - Y1r, 0.0)
    Ac = jnp.maximum(X2c - X1c, 0.0) * jnp.maximum(Y2c - Y1c, 0.0)

    def sup_mat(s, t):
        r = slice(s * _B, (s + 1) * _B)
        c = slice(t * _B, (t + 1) * _B)
        xx1 = jnp.maximum(X1r[r, :], X1c[:, c])
        yy1 = jnp.maximum(Y1r[r, :], Y1c[:, c])
        xx2 = jnp.minimum(X2r[r, :], X2c[:, c])
        yy2 = jnp.minimum(Y2r[r, :], Y2c[:, c])
        inter = jnp.maximum(xx2 - xx1, 0.0) * jnp.maximum(yy2 - yy1, 0.0)
        iou = inter / (Ar[r, :] + Ac[:, c] - inter + 1e-9)
        return jnp.where(iou > _IOU_THRESH, 1.0, 0.0)

    pos_iota = lax.broadcasted_iota(jnp.int32, (1, _P), 1)
    row_iota = lax.broadcasted_iota(jnp.int32, (_B, 1), 0)
    col_iota = lax.broadcasted_iota(jnp.int32, (1, _B), 1)
    tri = jnp.where(row_iota < col_iota, 1.0, 0.0)

    keep_blocks = []
    for t in range(_T):
        base = t * _B
        m = jnp.where(pos_iota[:, base:base + _B] < _N, 1.0, 0.0)
        for s in range(t):
            sup = jnp.dot(keep_blocks[s], sup_mat(s, t),
                          preferred_element_type=jnp.float32)
            m = jnp.where(sup > 0.0, 0.0, m)
        Stt = sup_mat(t, t) * tri

        def fix_cond(carry):
            return carry[1]

        def fix_body(carry):
            k = carry[0]
            kn = jnp.where(
                jnp.dot(k, Stt, preferred_element_type=jnp.float32) > 0.0,
                0.0, m)
            return kn, jnp.any(kn != k)

        k, _ = lax.while_loop(fix_cond, fix_body, (m, jnp.bool_(True)))
        keep_blocks.append(k)

    K = jnp.concatenate(keep_blocks, axis=1)
    valid = (K > 0.0) & (S >= _SCORE_THRESH)
    is_h = L == _HUMAN
    hum = jnp.where(valid & is_h, S, _NEG)
    obj = jnp.where(valid & (~is_h), S, _NEG)

    out_iota = lax.broadcasted_iota(jnp.int32, (1, 128), 1)
    res = jnp.zeros((1, 128), jnp.int32)

    def pick15(msc, res, off):
        for r in range(_MAXI):
            mx = jnp.max(msc)
            ind = jnp.min(jnp.where(msc == mx, pos_iota, _P))
            res = jnp.where(out_iota == off + r, ind, res)
            msc = jnp.where(pos_iota == ind, -jnp.inf, msc)
        return res

    res = pick15(hum, res, 0)
    res = pick15(obj, res, _MAXI)
    out_ref[...] = res


def kernel(boxes, scores, labels, hidden_states):
    max_coord = jnp.max(boxes) + 1.0
    off = labels.astype(boxes.dtype) * max_coord
    ob = boxes + off[:, None]
    order = jnp.argsort(-scores)
    sob = ob[order]
    ssc = scores[order]
    slb = labels[order]

    pad = _P - _N
    sob_p = jnp.pad(sob, ((0, pad), (0, 0)))
    ssc_p = jnp.pad(ssc, (0, pad))
    slb_p = jnp.pad(slb, (0, pad), constant_values=1)

    x1r = sob_p[:, 0:1]
    y1r = sob_p[:, 1:2]
    x2r = sob_p[:, 2:3]
    y2r = sob_p[:, 3:4]
    x1c = x1r.reshape(1, _P)
    y1c = y1r.reshape(1, _P)
    x2c = x2r.reshape(1, _P)
    y2c = y2r.reshape(1, _P)

    idx128 = pl.pallas_call(
        _nms_topk_kernel,
        out_shape=jax.ShapeDtypeStruct((1, 128), jnp.int32),
    )(x1r, y1r, x2r, y2r, x1c, y1c, x2c, y2c,
      ssc_p.reshape(1, _P), slb_p.reshape(1, _P))

    kidx = idx128[0, :30]
    g = order[kidx]
    return boxes[g], scores[g], labels[g], hidden_states[g]


# speedup vs baseline: 281.5358x; 1.0301x over previous
import jax
import jax.numpy as jnp
from jax import lax
from jax.experimental import pallas as pl

_N = 5000
_B = 512
_P = 5120
_T = _P // _B
_IOU_THRESH = 0.5
_SCORE_THRESH = 0.2
_NEG = -1e30
_MAXI = 15
_HUMAN = 0


def _nms_topk_kernel(x1r, y1r, x2r, y2r, x1c, y1c, x2c, y2c, sc, lb, out_ref):
    X1r, Y1r, X2r, Y2r = x1r[...], y1r[...], x2r[...], y2r[...]
    X1c, Y1c, X2c, Y2c = x1c[...], y1c[...], x2c[...], y2c[...]
    S = sc[...]
    L = lb[...]

    Ar = jnp.maximum(X2r - X1r, 0.0) * jnp.maximum(Y2r - Y1r, 0.0)
    Ac = jnp.maximum(X2c - X1c, 0.0) * jnp.maximum(Y2c - Y1c, 0.0)

    def sup_mat(s, t):
        r = slice(s * _B, (s + 1) * _B)
        c = slice(t * _B, (t + 1) * _B)
        xx1 = jnp.maximum(X1r[r, :], X1c[:, c])
        yy1 = jnp.maximum(Y1r[r, :], Y1c[:, c])
        xx2 = jnp.minimum(X2r[r, :], X2c[:, c])
        yy2 = jnp.minimum(Y2r[r, :], Y2c[:, c])
        inter = jnp.maximum(xx2 - xx1, 0.0) * jnp.maximum(yy2 - yy1, 0.0)
        iou = inter / (Ar[r, :] + Ac[:, c] - inter + 1e-9)
        return jnp.where(iou > _IOU_THRESH, 1.0, 0.0)

    pos_iota = lax.broadcasted_iota(jnp.int32, (1, _P), 1)
    row_iota = lax.broadcasted_iota(jnp.int32, (_B, 1), 0)
    col_iota = lax.broadcasted_iota(jnp.int32, (1, _B), 1)
    tri = jnp.where(row_iota < col_iota, 1.0, 0.0)

    n_above = jnp.sum(jnp.where(S >= _SCORE_THRESH, 1, 0))

    keep_blocks = []
    for t in range(_T):
        base = t * _B

        def compute_block(t=t, base=base, prev=tuple(keep_blocks)):
            m = jnp.where(pos_iota[:, base:base + _B] < _N, 1.0, 0.0)
            for s in range(t):
                sup = jnp.dot(prev[s], sup_mat(s, t),
                              preferred_element_type=jnp.float32)
                m = jnp.where(sup > 0.0, 0.0, m)
            Stt = sup_mat(t, t) * tri

            def fix_cond(carry):
                return carry[1]

            def fix_body(carry):
                k = carry[0]
                kn = jnp.where(
                    jnp.dot(k, Stt, preferred_element_type=jnp.float32)
                    > 0.0, 0.0, m)
                return kn, jnp.any(kn != k)

            k, _ = lax.while_loop(fix_cond, fix_body, (m, jnp.bool_(True)))
            return k

        k = lax.cond(base < n_above, compute_block,
                     lambda: jnp.zeros((1, _B), jnp.float32))
        keep_blocks.append(k)

    K = jnp.concatenate(keep_blocks, axis=1)
    valid = (K > 0.0) & (S >= _SCORE_THRESH)
    is_h = L == _HUMAN
    hum = jnp.where(valid & is_h, S, _NEG)
    obj = jnp.where(valid & (~is_h), S, _NEG)

    out_iota = lax.broadcasted_iota(jnp.int32, (1, 128), 1)
    res = jnp.zeros((1, 128), jnp.int32)

    def pick15(msc, res, off):
        for r in range(_MAXI):
            mx = jnp.max(msc)
            ind = jnp.min(jnp.where(msc == mx, pos_iota, _P))
            res = jnp.where(out_iota == off + r, ind, res)
            msc = jnp.where(pos_iota == ind, -jnp.inf, msc)
        return res

    res = pick15(hum, res, 0)
    res = pick15(obj, res, _MAXI)
    out_ref[...] = res


def kernel(boxes, scores, labels, hidden_states):
    max_coord = jnp.max(boxes) + 1.0
    off = labels.astype(boxes.dtype) * max_coord
    ob = boxes + off[:, None]
    order = jnp.argsort(-scores)
    sob = ob[order]
    ssc = scores[order]
    slb = labels[order]

    pad = _P - _N
    sob_p = jnp.pad(sob, ((0, pad), (0, 0)))
    ssc_p = jnp.pad(ssc, (0, pad))
    slb_p = jnp.pad(slb, (0, pad), constant_values=1)

    x1r = sob_p[:, 0:1]
    y1r = sob_p[:, 1:2]
    x2r = sob_p[:, 2:3]
    y2r = sob_p[:, 3:4]
    x1c = x1r.reshape(1, _P)
    y1c = y1r.reshape(1, _P)
    x2c = x2r.reshape(1, _P)
    y2c = y2r.reshape(1, _P)

    idx128 = pl.pallas_call(
        _nms_topk_kernel,
        out_shape=jax.ShapeDtypeStruct((1, 128), jnp.int32),
    )(x1r, y1r, x2r, y2r, x1c, y1c, x2c, y2c,
      ssc_p.reshape(1, _P), slb_p.reshape(1, _P))

    kidx = idx128[0, :30]
    g = order[kidx]
    return boxes[g], scores[g], labels[g], hidden_states[g]


# speedup vs baseline: 353.2407x; 1.2547x over previous
import jax
import jax.numpy as jnp
from jax import lax
from jax.experimental import pallas as pl

_N = 5000
_B = 512
_P = 5120
_T = _P // _B
_IOU_THRESH = 0.5
_SCORE_THRESH = 0.2
_NEG = -1e30
_MAXI = 15
_HUMAN = 0


def _nms_topk_kernel(x1r, y1r, x2r, y2r, x1c, y1c, x2c, y2c, sc, lbf, out_ref):
    X1r, Y1r, X2r, Y2r = x1r[...], y1r[...], x2r[...], y2r[...]
    X1c, Y1c, X2c, Y2c = x1c[...], y1c[...], x2c[...], y2c[...]
    S = sc[...]
    L = lbf[...]

    Ar = jnp.maximum(X2r - X1r, 0.0) * jnp.maximum(Y2r - Y1r, 0.0)
    Ac = jnp.maximum(X2c - X1c, 0.0) * jnp.maximum(Y2c - Y1c, 0.0)

    def sup_mat(s, t):
        r = slice(s * _B, (s + 1) * _B)
        c = slice(t * _B, (t + 1) * _B)
        xx1 = jnp.maximum(X1r[r, :], X1c[:, c])
        yy1 = jnp.maximum(Y1r[r, :], Y1c[:, c])
        xx2 = jnp.minimum(X2r[r, :], X2c[:, c])
        yy2 = jnp.minimum(Y2r[r, :], Y2c[:, c])
        inter = jnp.maximum(xx2 - xx1, 0.0) * jnp.maximum(yy2 - yy1, 0.0)
        iou = inter / (Ar[r, :] + Ac[:, c] - inter + 1e-9)
        return jnp.where(iou > _IOU_THRESH, 1.0, 0.0)

    pos_iota = lax.broadcasted_iota(jnp.int32, (1, _P), 1)
    row_iota = lax.broadcasted_iota(jnp.int32, (_B, 1), 0)
    col_iota = lax.broadcasted_iota(jnp.int32, (1, _B), 1)
    tri = jnp.where(row_iota < col_iota, 1.0, 0.0)

    n_above = jnp.sum(jnp.where(S >= _SCORE_THRESH, 1, 0))

    keep_blocks = []
    for t in range(_T):
        base = t * _B

        def compute_block(t=t, base=base, prev=tuple(keep_blocks)):
            m = jnp.where(pos_iota[:, base:base + _B] < _N, 1.0, 0.0)
            for s in range(t):
                sup = jnp.dot(prev[s], sup_mat(s, t),
                              preferred_element_type=jnp.float32)
                m = jnp.where(sup > 0.0, 0.0, m)
            Stt = sup_mat(t, t) * tri

            def fix_cond(carry):
                return carry[1]

            def fix_body(carry):
                k = carry[0]
                kn = jnp.where(
                    jnp.dot(k, Stt, preferred_element_type=jnp.float32)
                    > 0.0, 0.0, m)
                return kn, jnp.any(kn != k)

            k, _ = lax.while_loop(fix_cond, fix_body, (m, jnp.bool_(True)))
            return k

        k = lax.cond(base < n_above, compute_block,
                     lambda: jnp.zeros((1, _B), jnp.float32))
        keep_blocks.append(k)

    K = jnp.concatenate(keep_blocks, axis=1)
    valid = (K > 0.0) & (S >= _SCORE_THRESH)
    is_h = L == jnp.float32(_HUMAN)
    hum = jnp.where(valid & is_h, S, _NEG)
    obj = jnp.where(valid & (~is_h), S, _NEG)

    out_iota = lax.broadcasted_iota(jnp.int32, (1, 128), 1)
    res = jnp.zeros((1, 128), jnp.int32)

    def pick15(msc, res, off):
        for r in range(_MAXI):
            mx = jnp.max(msc)
            ind = jnp.min(jnp.where(msc == mx, pos_iota, _P))
            res = jnp.where(out_iota == off + r, ind, res)
            msc = jnp.where(pos_iota == ind, -jnp.inf, msc)
        return res

    res = pick15(hum, res, 0)
    res = pick15(obj, res, _MAXI)
    out_ref[...] = res


def kernel(boxes, scores, labels, hidden_states):
    max_coord = jnp.max(boxes) + 1.0
    lbf = labels.astype(boxes.dtype)
    cat = jnp.concatenate([boxes, scores[:, None], lbf[:, None]], axis=1)
    order = jnp.argsort(-scores)
    sg = cat[order]

    soff = sg[:, 5] * max_coord
    sob = sg[:, 0:4] + soff[:, None]

    pad = _P - _N
    sob_p = jnp.pad(sob, ((0, pad), (0, 0)))
    ssc_p = jnp.pad(sg[:, 4], (0, pad))
    slb_p = jnp.pad(sg[:, 5], (0, pad))

    x1r = sob_p[:, 0:1]
    y1r = sob_p[:, 1:2]
    x2r = sob_p[:, 2:3]
    y2r = sob_p[:, 3:4]
    x1c = x1r.reshape(1, _P)
    y1c = y1r.reshape(1, _P)
    x2c = x2r.reshape(1, _P)
    y2c = y2r.reshape(1, _P)

    idx128 = pl.pallas_call(
        _nms_topk_kernel,
        out_shape=jax.ShapeDtypeStruct((1, 128), jnp.int32),
    )(x1r, y1r, x2r, y2r, x1c, y1c, x2c, y2c,
      ssc_p.reshape(1, _P), slb_p.reshape(1, _P))

    kidx = idx128[0, :30]
    sel = sg[kidx]
    hs = hidden_states[order[kidx]]
    return (sel[:, 0:4], sel[:, 4], sel[:, 5].astype(labels.dtype), hs)
